# Initial kernel scaffold; baseline (speedup 1.0000x reference)
import functools

import jax
import jax.numpy as jnp
from jax import lax
from jax.experimental import pallas as pl
from jax.experimental.pallas import tpu as pltpu
from jax.experimental.pallas import tpu_sc as plsc

N = 1_000_000
D = 16
NC = 2
NS = 16
NW = NC * NS
R = N // NW
C = 1250
NCHUNK = R // C
GPC = (C + 15) // 16
CPAD = GPC * 16

_mesh = plsc.VectorSubcoreMesh(core_axis_name="c", subcore_axis_name="s")


@functools.partial(
    pl.kernel,
    out_type=(
        jax.ShapeDtypeStruct((NW * D,), jnp.float32),
        jax.ShapeDtypeStruct((NW * D,), jnp.int32),
    ),
    mesh=_mesh,
    scratch_types=(
        pltpu.VMEM((D,), jnp.float32),
        pltpu.VMEM((CPAD, D), jnp.float32),
        pltpu.VMEM((CPAD, D), jnp.float32),
        pltpu.VMEM((D,), jnp.float32),
        pltpu.VMEM((D,), jnp.int32),
        pltpu.SemaphoreType.DMA,
        pltpu.SemaphoreType.DMA,
    ),
)
def _nn_scan(query_hbm, keys_hbm, score_out, idx_out,
             qv, buf0, buf1, sbest, ibest, sem0, sem1):
    cid = lax.axis_index("c")
    sid = lax.axis_index("s")
    wid = sid * NC + cid
    row0 = wid * R

    pltpu.sync_copy(query_hbm, qv)
    qb = [jnp.broadcast_to(qv[d], (16,)) for d in range(D)]
    iota = lax.iota(jnp.int32, 16)
    cols = [jnp.full((16,), d, jnp.int32) for d in range(D)]

    bufs = (buf0, buf1)
    sems = (sem0, sem1)

    def start(j):
        return pltpu.async_copy(
            keys_hbm.at[pl.ds(row0 + j * C, C)],
            bufs[j % 2].at[pl.ds(0, C)],
            sems[j % 2],
        )

    pending = start(0)
    bs = jnp.full((16,), jnp.inf, jnp.float32)
    bi = jnp.zeros((16,), jnp.int32)

    for j in range(NCHUNK):
        pending.wait()
        if j + 1 < NCHUNK:
            pending = start(j + 1)
        cur = bufs[j % 2]
        gbase = jnp.full((16,), row0 + j * C, jnp.int32)

        def group_body(g, carry, cur=cur, gbase=gbase):
            bs, bi = carry
            ir = iota + g * 16
            acc = jnp.zeros((16,), jnp.float32)
            for d in range(D):
                col = plsc.load_gather(cur, [ir, cols[d]])
                t = col - qb[d]
                acc = acc + t * t
            valid = ir < C
            m = valid & (acc < bs)
            bs = jnp.where(m, acc, bs)
            bi = jnp.where(m, ir + gbase, bi)
            return bs, bi

        bs, bi = lax.fori_loop(0, GPC, group_body, (bs, bi))

    sbest[...] = bs
    ibest[...] = bi
    pltpu.sync_copy(sbest, score_out.at[pl.ds(wid * D, D)])
    pltpu.sync_copy(ibest, idx_out.at[pl.ds(wid * D, D)])


@functools.partial(
    pl.kernel,
    out_type=jax.ShapeDtypeStruct((16,), jnp.float32),
    mesh=_mesh,
    scratch_types=(
        pltpu.VMEM((NW * D,), jnp.float32),
        pltpu.VMEM((NW * D,), jnp.int32),
        pltpu.VMEM((D,), jnp.float32),
        pltpu.VMEM((16,), jnp.int32),
        pltpu.VMEM((16, D), jnp.float32),
        pltpu.VMEM((16,), jnp.float32),
        pltpu.SemaphoreType.DMA,
    ),
)
def _nn_finish(query_hbm, values_hbm, score_hbm, idx_hbm, out_hbm,
               sbuf, ibuf, qv, widx, vrow, ob, sem):
    cid = lax.axis_index("c")
    sid = lax.axis_index("s")

    @pl.when((cid == 0) & (sid == 0))
    def _():
        pltpu.sync_copy(score_hbm, sbuf)
        pltpu.sync_copy(idx_hbm, ibuf)
        pltpu.sync_copy(query_hbm, qv)
        bs = sbuf[pl.ds(0, 16)]
        bi = ibuf[pl.ds(0, 16)]
        for r in range(1, NW):
            s = sbuf[pl.ds(r * 16, 16)]
            i = ibuf[pl.ds(r * 16, 16)]
            m = s < bs
            bs = jnp.where(m, s, bs)
            bi = jnp.where(m, i, bi)
        _, sv = plsc.sort_key_val(bs, bi)
        widx[...] = sv
        pltpu.async_copy(values_hbm.at[widx], vrow, sem).wait()
        p = vrow[0] * qv[...]
        ob[...] = jnp.broadcast_to(jnp.sum(p), (16,))
        pltpu.sync_copy(ob, out_hbm)


def kernel(query, keys, values):
    scores, ids = _nn_scan(query, keys)
    out16 = _nn_finish(query, values, scores, ids)
    return out16[:1]


# speedup vs baseline: 2.2339x; 2.2339x over previous
import functools

import jax
import jax.numpy as jnp
from jax import lax
from jax.experimental import pallas as pl
from jax.experimental.pallas import tpu as pltpu
from jax.experimental.pallas import tpu_sc as plsc

N = 1_000_000
D = 16
NC = 2
NS = 16
NW = NC * NS
C = 400
G = C // 16
NCHUNK = N // C
TFULL = NCHUNK // NW
NEXTRA = NCHUNK - TFULL * NW

_mesh = plsc.VectorSubcoreMesh(core_axis_name="c", subcore_axis_name="s")


@functools.partial(
    pl.kernel,
    out_type=(
        jax.ShapeDtypeStruct((NW * 16,), jnp.float32),
        jax.ShapeDtypeStruct((NW * 16,), jnp.int32),
    ),
    mesh=_mesh,
    compiler_params=pltpu.CompilerParams(needs_layout_passes=False),
    scratch_types=(
        pltpu.VMEM((D,), jnp.float32),
        pltpu.VMEM((C, D), jnp.float32),
        pltpu.VMEM((C, D), jnp.float32),
        pltpu.VMEM((16,), jnp.float32),
        pltpu.VMEM((16,), jnp.int32),
        pltpu.SemaphoreType.DMA,
        pltpu.SemaphoreType.DMA,
    ),
)
def _nn_scan(query_hbm, keys_hbm, score_out, idx_out,
             qv, buf0, buf1, sbest, ibest, sem0, sem1):
    cid = lax.axis_index("c")
    sid = lax.axis_index("s")
    wid = sid * NC + cid

    pltpu.sync_copy(query_hbm, qv)
    q = qv[...]
    qb = [jnp.broadcast_to(q[d], (16,)) for d in range(D)]
    iota = lax.iota(jnp.int32, 16)
    cols = [jnp.full((16,), d, jnp.int32) for d in range(D)]

    def start(t, buf, sem):
        row = pl.multiple_of((wid + NW * t) * C, 16)
        return pltpu.async_copy(keys_hbm.at[pl.ds(row, C)], buf, sem)

    def compute(t, buf, bs, bi):
        gbase = (wid + NW * t) * C

        def group_body(g, carry):
            bs, bi = carry
            ir = iota + g * 16
            acc = jnp.zeros((16,), jnp.float32)
            for d in range(D):
                col = plsc.load_gather(buf, [ir, cols[d]])
                t_ = col - qb[d]
                acc = acc + t_ * t_
            m = acc < bs
            bs = jnp.where(m, acc, bs)
            bi = jnp.where(m, ir + gbase, bi)
            return bs, bi

        return lax.fori_loop(0, G, group_body, (bs, bi))

    start(0, buf0, sem0)
    start(1, buf1, sem1)
    bs0 = jnp.full((16,), jnp.inf, jnp.float32)
    bi0 = jnp.zeros((16,), jnp.int32)
    extra = wid < NEXTRA

    def round_body(tt, carry):
        bs, bi = carry
        t0 = 2 * tt
        pltpu.make_async_copy(keys_hbm.at[pl.ds(0, C)], buf0, sem0).wait()
        bs, bi = compute(t0, buf0, bs, bi)

        @pl.when((t0 + 2 < TFULL) | extra)
        def _():
            start(t0 + 2, buf0, sem0)

        pltpu.make_async_copy(keys_hbm.at[pl.ds(0, C)], buf1, sem1).wait()
        bs, bi = compute(t0 + 1, buf1, bs, bi)

        @pl.when(t0 + 3 < TFULL)
        def _():
            start(t0 + 3, buf1, sem1)

        return bs, bi

    bs, bi = lax.fori_loop(0, TFULL // 2, round_body, (bs0, bi0))

    @pl.when(extra)
    def _():
        pltpu.make_async_copy(keys_hbm.at[pl.ds(0, C)], buf0, sem0).wait()
        ebs, ebi = compute(TFULL, buf0, bs, bi)
        sbest[...] = ebs
        ibest[...] = ebi

    @pl.when(jnp.logical_not(extra))
    def _():
        sbest[...] = bs
        ibest[...] = bi

    pltpu.sync_copy(sbest, score_out.at[pl.ds(wid * 16, 16)])
    pltpu.sync_copy(ibest, idx_out.at[pl.ds(wid * 16, 16)])


@functools.partial(
    pl.kernel,
    out_type=jax.ShapeDtypeStruct((16,), jnp.float32),
    mesh=_mesh,
    compiler_params=pltpu.CompilerParams(needs_layout_passes=False),
    scratch_types=(
        pltpu.VMEM((NW * 16,), jnp.float32),
        pltpu.VMEM((NW * 16,), jnp.int32),
        pltpu.VMEM((D,), jnp.float32),
        pltpu.VMEM((8, D), jnp.float32),
        pltpu.VMEM((16,), jnp.float32),
        pltpu.SemaphoreType.DMA,
    ),
)
def _nn_finish(query_hbm, values_hbm, score_hbm, idx_hbm, out_hbm,
               sbuf, ibuf, qv, vblk, ob, sem):
    cid = lax.axis_index("c")
    sid = lax.axis_index("s")

    @pl.when((cid == 0) & (sid == 0))
    def _():
        pltpu.sync_copy(score_hbm, sbuf)
        pltpu.sync_copy(idx_hbm, ibuf)
        pltpu.sync_copy(query_hbm, qv)
        bs = sbuf[pl.ds(0, 16)]
        bi = ibuf[pl.ds(0, 16)]
        for r in range(1, NW):
            s = sbuf[pl.ds(r * 16, 16)]
            i = ibuf[pl.ds(r * 16, 16)]
            m = s < bs
            bs = jnp.where(m, s, bs)
            bi = jnp.where(m, i, bi)
        minv = jnp.min(bs)
        rid = jnp.min(jnp.where(bs == minv, bi, jnp.int32(2**31 - 1)))
        base = pl.multiple_of((rid // 8) * 8, 8)
        sub = jnp.broadcast_to(rid - base, (16,))
        pltpu.async_copy(values_hbm.at[pl.ds(base, 8)], vblk, sem).wait()
        row = jnp.zeros((16,), jnp.float32)
        for r in range(8):
            row = jnp.where(sub == r, vblk[r], row)
        p = row * qv[...]
        ob[...] = jnp.broadcast_to(jnp.sum(p), (16,))
        pltpu.sync_copy(ob, out_hbm)


def kernel(query, keys, values):
    scores, ids = _nn_scan(query, keys)
    out16 = _nn_finish(query, values, scores, ids)
    return out16[:1]
